# baseline (speedup 1.0000x reference)
import functools

import jax
import jax.numpy as jnp
from jax import lax
from jax.experimental import pallas as pl
from jax.experimental.pallas import tpu as pltpu
from jax.experimental.pallas import tpu_sc as plsc

N_NODES = 10000
D = 128
DH = 64
NT = 16
CHUNK = 128
ACC_ROWS = 10112
RPT = ACC_ROWS // NT
SENT = 10048
ROW_BLK = 1000

_SC_PARAMS = pltpu.CompilerParams(use_tc_tiling_on_sc=False)


def _make_agg(n_chunks, with_deg):
  mesh = plsc.VectorSubcoreMesh(core_axis_name="c", subcore_axis_name="s")

  out_type = [
      jax.ShapeDtypeStruct((ACC_ROWS, DH), jnp.float32),
      jax.ShapeDtypeStruct((ACC_ROWS, DH), jnp.float32),
  ]
  scratch = [
      pltpu.VMEM_SHARED((ACC_ROWS, DH), jnp.float32),
      pltpu.VMEM_SHARED((N_NODES, DH), jnp.float32),
      pltpu.VMEM((n_chunks // 2, CHUNK), jnp.int32),
      pltpu.VMEM((n_chunks // 2, CHUNK), jnp.int32),
      pltpu.VMEM((CHUNK, DH), jnp.float32),
      pltpu.VMEM((CHUNK, DH), jnp.float32),
      pltpu.SemaphoreType.DMA,
      pltpu.SemaphoreType.DMA,
  ]
  if with_deg:
    out_type += [
        jax.ShapeDtypeStruct((ACC_ROWS, 16), jnp.float32),
        jax.ShapeDtypeStruct((ACC_ROWS, 16), jnp.float32),
    ]
    scratch += [
        pltpu.VMEM_SHARED((ACC_ROWS, 16), jnp.float32),
        pltpu.VMEM((CHUNK, 16), jnp.float32),
    ]

  @functools.partial(
      pl.kernel,
      out_type=out_type,
      mesh=mesh,
      compiler_params=_SC_PARAMS,
      scratch_types=scratch,
  )
  def agg(y_lo, y_hi, src3, dst3, zeros_hbm, *rest):
    if with_deg:
      (ones_hbm, zeros16_hbm, out_lo, out_hi, deg0, deg1,
       acc, y_spm, sidx, didx, r0, r1, g0, g1, deg_sh, ones_v) = rest
    else:
      (out_lo, out_hi, acc, y_spm, sidx, didx, r0, r1, g0, g1) = rest
    c = lax.axis_index("c")
    s = lax.axis_index("s")
    rows = [r0, r1]
    gsem = [g0, g1]
    hc = n_chunks // 2
    ypt = N_NODES // NT

    def run(table, out):
      pltpu.sync_copy(table.at[pl.ds(s * ypt, ypt)],
                      y_spm.at[pl.ds(s * ypt, ypt)])
      pltpu.sync_copy(zeros_hbm, acc.at[pl.ds(s * RPT, RPT)])
      if with_deg:
        pltpu.sync_copy(ones_hbm, ones_v)
        pltpu.sync_copy(zeros16_hbm, deg_sh.at[pl.ds(s * RPT, RPT)])
      plsc.subcore_barrier()

      def phase(p, carry):
        pltpu.sync_copy(src3.at[s, pl.ds(p * hc, hc)], sidx)
        pltpu.sync_copy(dst3.at[s, pl.ds(p * hc, hc)], didx)
        pltpu.async_copy(y_spm.at[sidx.at[0]], rows[0], gsem[0])

        def body(j, carry):
          even = j % 2 == 0

          @pl.when(jnp.logical_and(even, j + 1 < hc))
          def _():
            pltpu.async_copy(y_spm.at[sidx.at[j + 1]], rows[1], gsem[1])

          @pl.when(jnp.logical_and(~even, j + 1 < hc))
          def _():
            pltpu.async_copy(y_spm.at[sidx.at[j + 1]], rows[0], gsem[0])

          if with_deg:
            @pl.when(p == c)
            def _():
              pltpu.sync_copy(ones_v, deg_sh.at[didx.at[j]], add=True)

          @pl.when(even)
          def _():
            pltpu.make_async_copy(
                y_spm.at[sidx.at[0]], rows[0], gsem[0]).wait()
            pltpu.sync_copy(rows[0], acc.at[didx.at[j]], add=True)

          @pl.when(~even)
          def _():
            pltpu.make_async_copy(
                y_spm.at[sidx.at[0]], rows[1], gsem[1]).wait()
            pltpu.sync_copy(rows[1], acc.at[didx.at[j]], add=True)

          return carry

        lax.fori_loop(0, hc, body, 0)
        return carry

      lax.fori_loop(0, 2, phase, 0)
      plsc.subcore_barrier()

      pltpu.sync_copy(acc.at[pl.ds(s * RPT, RPT)],
                      out.at[pl.ds(s * RPT, RPT)])
      if with_deg:
        @pl.when(c == 0)
        def _():
          pltpu.sync_copy(deg_sh.at[pl.ds(s * RPT, RPT)],
                          deg0.at[pl.ds(s * RPT, RPT)])

        @pl.when(c == 1)
        def _():
          pltpu.sync_copy(deg_sh.at[pl.ds(s * RPT, RPT)],
                          deg1.at[pl.ds(s * RPT, RPT)])

    @pl.when(c == 0)
    def _():
      run(y_lo, out_lo)

    @pl.when(c == 1)
    def _():
      run(y_hi, out_hi)

  return agg


def _row_spec(cols):
  return pl.BlockSpec((ROW_BLK, cols), lambda i: (i, 0))


def _full_spec(shape):
  return pl.BlockSpec(shape, lambda i: (0,) * len(shape))


def _tc_first(x, w_self, w_neigh, b):
  def body(x_ref, ws_ref, wn_ref, b_ref, s_ref, ylo_ref, yhi_ref):
    x = x_ref[...]
    s_ref[...] = (
        jnp.dot(x, ws_ref[...], preferred_element_type=jnp.float32)
        + b_ref[...])
    y = jnp.dot(x, wn_ref[...], preferred_element_type=jnp.float32)
    ylo_ref[...] = y[:, :DH]
    yhi_ref[...] = y[:, DH:]

  n = x.shape[0]
  return pl.pallas_call(
      body,
      grid=(n // ROW_BLK,),
      in_specs=[_row_spec(D), _full_spec((D, D)), _full_spec((D, D)),
                _full_spec((1, D))],
      out_specs=[_row_spec(D), _row_spec(DH), _row_spec(DH)],
      out_shape=[
          jax.ShapeDtypeStruct((n, D), jnp.float32),
          jax.ShapeDtypeStruct((n, DH), jnp.float32),
          jax.ShapeDtypeStruct((n, DH), jnp.float32),
      ],
  )(x, w_self, w_neigh, b)


def _tc_mid(s_prev, a_lo, a_hi, deg, w_self, w_neigh, b):
  def body(s_ref, alo_ref, ahi_ref, deg_ref, ws_ref, wn_ref, b_ref,
           so_ref, ylo_ref, yhi_ref):
    inv = 1.0 / jnp.maximum(deg_ref[...], 1.0)
    a = jnp.concatenate([alo_ref[...], ahi_ref[...]], axis=1) * inv
    h = jnp.maximum(s_ref[...] + a, 0.0)
    so_ref[...] = (
        jnp.dot(h, ws_ref[...], preferred_element_type=jnp.float32)
        + b_ref[...])
    y = jnp.dot(h, wn_ref[...], preferred_element_type=jnp.float32)
    ylo_ref[...] = y[:, :DH]
    yhi_ref[...] = y[:, DH:]

  n = s_prev.shape[0]
  return pl.pallas_call(
      body,
      grid=(n // ROW_BLK,),
      in_specs=[_row_spec(D), _row_spec(DH), _row_spec(DH), _row_spec(1),
                _full_spec((D, D)), _full_spec((D, D)), _full_spec((1, D))],
      out_specs=[_row_spec(D), _row_spec(DH), _row_spec(DH)],
      out_shape=[
          jax.ShapeDtypeStruct((n, D), jnp.float32),
          jax.ShapeDtypeStruct((n, DH), jnp.float32),
          jax.ShapeDtypeStruct((n, DH), jnp.float32),
      ],
  )(s_prev, a_lo, a_hi, deg, w_self, w_neigh, b)


def _tc_final(s_prev, a_lo, a_hi, deg):
  def body(s_ref, alo_ref, ahi_ref, deg_ref, out_ref):
    inv = 1.0 / jnp.maximum(deg_ref[...], 1.0)
    a = jnp.concatenate([alo_ref[...], ahi_ref[...]], axis=1) * inv
    out_ref[...] = s_ref[...] + a

  n = s_prev.shape[0]
  return pl.pallas_call(
      body,
      grid=(n // ROW_BLK,),
      in_specs=[_row_spec(D), _row_spec(DH), _row_spec(DH), _row_spec(1)],
      out_specs=_row_spec(D),
      out_shape=jax.ShapeDtypeStruct((n, D), jnp.float32),
  )(s_prev, a_lo, a_hi, deg)


@jax.jit
def kernel(in_feat, edge_index, W_self0, W_neigh0, b0,
           W_self1, W_neigh1, b1, W_self2, W_neigh2, b2):
  n, _ = in_feat.shape
  e = edge_index.shape[1]

  per_tile = -(-e // NT)
  n_chunks = -(-per_tile // CHUNK)
  n_chunks = -(-n_chunks // 4) * 4
  e_pad = NT * n_chunks * CHUNK
  src = jnp.concatenate(
      [edge_index[0], jnp.zeros((e_pad - e,), jnp.int32)])
  dst = jnp.concatenate(
      [edge_index[1], jnp.full((e_pad - e,), SENT, jnp.int32)])
  src3 = src.reshape(NT, n_chunks, CHUNK)
  dst3 = dst.reshape(NT, n_chunks, CHUNK)

  zeros_hbm = jnp.zeros((RPT, DH), jnp.float32)
  zeros16 = jnp.zeros((RPT, 16), jnp.float32)
  ones_hbm = jnp.ones((CHUNK, 16), jnp.float32)

  agg_deg = _make_agg(n_chunks, True)
  agg = _make_agg(n_chunks, False)

  s0, ylo, yhi = _tc_first(in_feat, W_self0, W_neigh0, b0.reshape(1, D))
  alo, ahi, dg0, dg1 = agg_deg(ylo, yhi, src3, dst3, zeros_hbm,
                               ones_hbm, zeros16)
  deg = (dg0 + dg1)[:n, 0:1]
  s1, ylo, yhi = _tc_mid(s0, alo[:n], ahi[:n], deg,
                         W_self1, W_neigh1, b1.reshape(1, D))
  alo, ahi = agg(ylo, yhi, src3, dst3, zeros_hbm)
  s2, ylo, yhi = _tc_mid(s1, alo[:n], ahi[:n], deg,
                         W_self2, W_neigh2, b2.reshape(1, D))
  alo, ahi = agg(ylo, yhi, src3, dst3, zeros_hbm)
  return _tc_final(s2, alo[:n], ahi[:n], deg)


# speedup vs baseline: 1.0620x; 1.0620x over previous
import functools

import jax
import jax.numpy as jnp
from jax import lax
from jax.experimental import pallas as pl
from jax.experimental.pallas import tpu as pltpu
from jax.experimental.pallas import tpu_sc as plsc

N_NODES = 10000
D = 128
DH = 64
NT = 16
CHUNK = 128
ACC_ROWS = 10112
RPT = ACC_ROWS // NT
SENT = 10048
ROW_BLK = 1000

_SC_PARAMS = pltpu.CompilerParams(use_tc_tiling_on_sc=False)


def _make_agg(n_chunks):
  mesh = plsc.VectorSubcoreMesh(core_axis_name="c", subcore_axis_name="s")

  @functools.partial(
      pl.kernel,
      out_type=[
          jax.ShapeDtypeStruct((ACC_ROWS, DH), jnp.float32),
          jax.ShapeDtypeStruct((ACC_ROWS, DH), jnp.float32),
      ],
      mesh=mesh,
      compiler_params=_SC_PARAMS,
      scratch_types=[
          pltpu.VMEM_SHARED((ACC_ROWS, DH), jnp.float32),
          pltpu.VMEM_SHARED((N_NODES, DH), jnp.float32),
          pltpu.VMEM((n_chunks // 2, CHUNK), jnp.int32),
          pltpu.VMEM((n_chunks // 2, CHUNK), jnp.int32),
          pltpu.VMEM((CHUNK, DH), jnp.float32),
          pltpu.VMEM((CHUNK, DH), jnp.float32),
          pltpu.SemaphoreType.DMA,
          pltpu.SemaphoreType.DMA,
      ],
  )
  def agg(y_lo, y_hi, src3, dst3, zeros_hbm, out_lo, out_hi,
          acc, y_spm, sidx, didx, r0, r1, g0, g1):
    c = lax.axis_index("c")
    s = lax.axis_index("s")
    rows = [r0, r1]
    gsem = [g0, g1]
    hc = n_chunks // 2
    ypt = N_NODES // NT

    def run(table, out):
      pltpu.sync_copy(table.at[pl.ds(s * ypt, ypt)],
                      y_spm.at[pl.ds(s * ypt, ypt)])
      pltpu.sync_copy(zeros_hbm, acc.at[pl.ds(s * RPT, RPT)])
      plsc.subcore_barrier()

      def phase(p, carry):
        pltpu.sync_copy(src3.at[s, pl.ds(p * hc, hc)], sidx)
        pltpu.sync_copy(dst3.at[s, pl.ds(p * hc, hc)], didx)
        pltpu.async_copy(y_spm.at[sidx.at[0]], rows[0], gsem[0])

        def body(j, carry):
          even = j % 2 == 0

          @pl.when(jnp.logical_and(even, j + 1 < hc))
          def _():
            pltpu.async_copy(y_spm.at[sidx.at[j + 1]], rows[1], gsem[1])

          @pl.when(jnp.logical_and(~even, j + 1 < hc))
          def _():
            pltpu.async_copy(y_spm.at[sidx.at[j + 1]], rows[0], gsem[0])

          @pl.when(even)
          def _():
            pltpu.make_async_copy(
                y_spm.at[sidx.at[0]], rows[0], gsem[0]).wait()
            pltpu.sync_copy(rows[0], acc.at[didx.at[j]], add=True)

          @pl.when(~even)
          def _():
            pltpu.make_async_copy(
                y_spm.at[sidx.at[0]], rows[1], gsem[1]).wait()
            pltpu.sync_copy(rows[1], acc.at[didx.at[j]], add=True)

          return carry

        lax.fori_loop(0, hc, body, 0)
        return carry

      lax.fori_loop(0, 2, phase, 0)
      plsc.subcore_barrier()

      pltpu.sync_copy(acc.at[pl.ds(s * RPT, RPT)],
                      out.at[pl.ds(s * RPT, RPT)])

    @pl.when(c == 0)
    def _():
      run(y_lo, out_lo)

    @pl.when(c == 1)
    def _():
      run(y_hi, out_hi)

  return agg


def _make_deg(n_chunks):
  mesh = plsc.VectorSubcoreMesh(core_axis_name="c", subcore_axis_name="s")

  @functools.partial(
      pl.kernel,
      out_type=[
          jax.ShapeDtypeStruct((ACC_ROWS, 16), jnp.float32),
          jax.ShapeDtypeStruct((ACC_ROWS, 16), jnp.float32),
      ],
      mesh=mesh,
      compiler_params=_SC_PARAMS,
      scratch_types=[
          pltpu.VMEM_SHARED((ACC_ROWS, 16), jnp.float32),
          pltpu.VMEM((n_chunks // 2, CHUNK), jnp.int32),
          pltpu.VMEM((CHUNK, 16), jnp.float32),
      ],
  )
  def deg_kernel(dst3, ones_hbm, zeros16_hbm, deg0, deg1,
                 deg_sh, didx, ones_v):
    c = lax.axis_index("c")
    s = lax.axis_index("s")
    hc = n_chunks // 2

    pltpu.sync_copy(dst3.at[s, pl.ds(c * hc, hc)], didx)
    pltpu.sync_copy(ones_hbm, ones_v)
    pltpu.sync_copy(zeros16_hbm, deg_sh.at[pl.ds(s * RPT, RPT)])
    plsc.subcore_barrier()

    def body(j, carry):
      pltpu.sync_copy(ones_v, deg_sh.at[didx.at[j]], add=True)
      return carry

    lax.fori_loop(0, hc, body, 0)
    plsc.subcore_barrier()

    @pl.when(c == 0)
    def _():
      pltpu.sync_copy(deg_sh.at[pl.ds(s * RPT, RPT)],
                      deg0.at[pl.ds(s * RPT, RPT)])

    @pl.when(c == 1)
    def _():
      pltpu.sync_copy(deg_sh.at[pl.ds(s * RPT, RPT)],
                      deg1.at[pl.ds(s * RPT, RPT)])

  return deg_kernel


def _row_spec(cols):
  return pl.BlockSpec((ROW_BLK, cols), lambda i: (i, 0))


def _full_spec(shape):
  return pl.BlockSpec(shape, lambda i: (0,) * len(shape))


def _tc_first(x, w_self, w_neigh, b):
  def body(x_ref, ws_ref, wn_ref, b_ref, s_ref, ylo_ref, yhi_ref):
    x = x_ref[...]
    s_ref[...] = (
        jnp.dot(x, ws_ref[...], preferred_element_type=jnp.float32)
        + b_ref[...])
    y = jnp.dot(x, wn_ref[...], preferred_element_type=jnp.float32)
    ylo_ref[...] = y[:, :DH]
    yhi_ref[...] = y[:, DH:]

  n = x.shape[0]
  return pl.pallas_call(
      body,
      grid=(n // ROW_BLK,),
      in_specs=[_row_spec(D), _full_spec((D, D)), _full_spec((D, D)),
                _full_spec((1, D))],
      out_specs=[_row_spec(D), _row_spec(DH), _row_spec(DH)],
      out_shape=[
          jax.ShapeDtypeStruct((n, D), jnp.float32),
          jax.ShapeDtypeStruct((n, DH), jnp.float32),
          jax.ShapeDtypeStruct((n, DH), jnp.float32),
      ],
  )(x, w_self, w_neigh, b)


def _tc_mid(s_prev, a_lo, a_hi, dg0, dg1, w_self, w_neigh, b):
  def body(s_ref, alo_ref, ahi_ref, dg0_ref, dg1_ref, ws_ref, wn_ref, b_ref,
           so_ref, ylo_ref, yhi_ref):
    deg = dg0_ref[...][:, 0:1] + dg1_ref[...][:, 0:1]
    inv = 1.0 / jnp.maximum(deg, 1.0)
    a = jnp.concatenate([alo_ref[...], ahi_ref[...]], axis=1) * inv
    h = jnp.maximum(s_ref[...] + a, 0.0)
    so_ref[...] = (
        jnp.dot(h, ws_ref[...], preferred_element_type=jnp.float32)
        + b_ref[...])
    y = jnp.dot(h, wn_ref[...], preferred_element_type=jnp.float32)
    ylo_ref[...] = y[:, :DH]
    yhi_ref[...] = y[:, DH:]

  n = s_prev.shape[0]
  return pl.pallas_call(
      body,
      grid=(n // ROW_BLK,),
      in_specs=[_row_spec(D), _row_spec(DH), _row_spec(DH), _row_spec(16),
                _row_spec(16), _full_spec((D, D)), _full_spec((D, D)),
                _full_spec((1, D))],
      out_specs=[_row_spec(D), _row_spec(DH), _row_spec(DH)],
      out_shape=[
          jax.ShapeDtypeStruct((n, D), jnp.float32),
          jax.ShapeDtypeStruct((n, DH), jnp.float32),
          jax.ShapeDtypeStruct((n, DH), jnp.float32),
      ],
  )(s_prev, a_lo, a_hi, dg0, dg1, w_self, w_neigh, b)


def _tc_final(s_prev, a_lo, a_hi, dg0, dg1):
  def body(s_ref, alo_ref, ahi_ref, dg0_ref, dg1_ref, out_ref):
    deg = dg0_ref[...][:, 0:1] + dg1_ref[...][:, 0:1]
    inv = 1.0 / jnp.maximum(deg, 1.0)
    a = jnp.concatenate([alo_ref[...], ahi_ref[...]], axis=1) * inv
    out_ref[...] = s_ref[...] + a

  n = s_prev.shape[0]
  return pl.pallas_call(
      body,
      grid=(n // ROW_BLK,),
      in_specs=[_row_spec(D), _row_spec(DH), _row_spec(DH), _row_spec(16),
                _row_spec(16)],
      out_specs=_row_spec(D),
      out_shape=jax.ShapeDtypeStruct((n, D), jnp.float32),
  )(s_prev, a_lo, a_hi, dg0, dg1)


@jax.jit
def kernel(in_feat, edge_index, W_self0, W_neigh0, b0,
           W_self1, W_neigh1, b1, W_self2, W_neigh2, b2):
  n, _ = in_feat.shape
  e = edge_index.shape[1]

  per_tile = -(-e // NT)
  n_chunks = -(-per_tile // CHUNK)
  n_chunks = -(-n_chunks // 4) * 4
  e_pad = NT * n_chunks * CHUNK
  src = jnp.concatenate(
      [edge_index[0], jnp.zeros((e_pad - e,), jnp.int32)])
  dst = jnp.concatenate(
      [edge_index[1], jnp.full((e_pad - e,), SENT, jnp.int32)])
  src3 = src.reshape(NT, n_chunks, CHUNK)
  dst3 = dst.reshape(NT, n_chunks, CHUNK)

  zeros_hbm = jnp.zeros((RPT, DH), jnp.float32)
  zeros16 = jnp.zeros((RPT, 16), jnp.float32)
  ones_hbm = jnp.ones((CHUNK, 16), jnp.float32)

  agg = _make_agg(n_chunks)
  deg_kernel = _make_deg(n_chunks)

  dg0, dg1 = deg_kernel(dst3, ones_hbm, zeros16)

  s0, ylo, yhi = _tc_first(in_feat, W_self0, W_neigh0, b0.reshape(1, D))
  alo, ahi = agg(ylo, yhi, src3, dst3, zeros_hbm)
  s1, ylo, yhi = _tc_mid(s0, alo, ahi, dg0, dg1,
                         W_self1, W_neigh1, b1.reshape(1, D))
  alo, ahi = agg(ylo, yhi, src3, dst3, zeros_hbm)
  s2, ylo, yhi = _tc_mid(s1, alo, ahi, dg0, dg1,
                         W_self2, W_neigh2, b2.reshape(1, D))
  alo, ahi = agg(ylo, yhi, src3, dst3, zeros_hbm)
  return _tc_final(s2, alo, ahi, dg0, dg1)


# speedup vs baseline: 1.0807x; 1.0175x over previous
import functools

import jax
import jax.numpy as jnp
from jax import lax
from jax.experimental import pallas as pl
from jax.experimental.pallas import tpu as pltpu
from jax.experimental.pallas import tpu_sc as plsc

N_NODES = 10000
D = 128
DH = 64
NT = 16
CHUNK = 128
ACC_ROWS = 10112
RPT = ACC_ROWS // NT
SENT = 10048
ROW_BLK = 1000

_SC_PARAMS = pltpu.CompilerParams(use_tc_tiling_on_sc=False)


def _make_agg(n_chunks):
  mesh = plsc.VectorSubcoreMesh(core_axis_name="c", subcore_axis_name="s")

  @functools.partial(
      pl.kernel,
      out_type=[
          jax.ShapeDtypeStruct((ACC_ROWS, DH), jnp.float32),
          jax.ShapeDtypeStruct((ACC_ROWS, DH), jnp.float32),
      ],
      mesh=mesh,
      compiler_params=_SC_PARAMS,
      scratch_types=[
          pltpu.VMEM_SHARED((ACC_ROWS, DH), jnp.float32),
          pltpu.VMEM_SHARED((N_NODES, DH), jnp.float32),
          pltpu.VMEM((n_chunks // 2, CHUNK), jnp.int32),
          pltpu.VMEM((n_chunks // 2, CHUNK), jnp.int32),
          pltpu.VMEM((CHUNK, DH), jnp.float32),
          pltpu.VMEM((CHUNK, DH), jnp.float32),
          pltpu.SemaphoreType.DMA,
          pltpu.SemaphoreType.DMA,
          pltpu.SemaphoreType.DMA,
          pltpu.SemaphoreType.DMA,
      ],
  )
  def agg(y_lo, y_hi, src3, dst3, zeros_hbm, out_lo, out_hi,
          acc, y_spm, sidx, didx, r0, r1, g0, g1, u0, u1):
    c = lax.axis_index("c")
    s = lax.axis_index("s")
    rows = [r0, r1]
    gsem = [g0, g1]
    hc = n_chunks // 2
    ypt = N_NODES // NT

    def run(table, out):
      cp_y = pltpu.async_copy(table.at[pl.ds(s * ypt, ypt)],
                              y_spm.at[pl.ds(s * ypt, ypt)], u0)
      cp_z = pltpu.async_copy(zeros_hbm, acc.at[pl.ds(s * RPT, RPT)], u1)
      cp_s = pltpu.async_copy(src3.at[s, pl.ds(0, hc)], sidx, g0)
      cp_d = pltpu.async_copy(dst3.at[s, pl.ds(0, hc)], didx, g1)
      cp_y.wait()
      cp_z.wait()
      cp_s.wait()
      cp_d.wait()
      plsc.subcore_barrier()

      def phase(p):
        pltpu.async_copy(y_spm.at[sidx.at[0]], rows[0], gsem[0])

        def body(j, carry):
          even = j % 2 == 0

          @pl.when(jnp.logical_and(even, j + 1 < hc))
          def _():
            pltpu.async_copy(y_spm.at[sidx.at[j + 1]], rows[1], gsem[1])

          @pl.when(jnp.logical_and(~even, j + 1 < hc))
          def _():
            pltpu.async_copy(y_spm.at[sidx.at[j + 1]], rows[0], gsem[0])

          @pl.when(even)
          def _():
            pltpu.make_async_copy(
                y_spm.at[sidx.at[0]], rows[0], gsem[0]).wait()
            pltpu.sync_copy(rows[0], acc.at[didx.at[j]], add=True)

          @pl.when(~even)
          def _():
            pltpu.make_async_copy(
                y_spm.at[sidx.at[0]], rows[1], gsem[1]).wait()
            pltpu.sync_copy(rows[1], acc.at[didx.at[j]], add=True)

          return carry

        lax.fori_loop(0, hc, body, 0)

      phase(0)
      pltpu.sync_copy(src3.at[s, pl.ds(hc, hc)], sidx)
      pltpu.sync_copy(dst3.at[s, pl.ds(hc, hc)], didx)
      phase(1)
      plsc.subcore_barrier()

      pltpu.sync_copy(acc.at[pl.ds(s * RPT, RPT)],
                      out.at[pl.ds(s * RPT, RPT)])

    @pl.when(c == 0)
    def _():
      run(y_lo, out_lo)

    @pl.when(c == 1)
    def _():
      run(y_hi, out_hi)

  return agg


def _make_deg(n_chunks):
  mesh = plsc.VectorSubcoreMesh(core_axis_name="c", subcore_axis_name="s")

  @functools.partial(
      pl.kernel,
      out_type=[
          jax.ShapeDtypeStruct((ACC_ROWS, 16), jnp.float32),
          jax.ShapeDtypeStruct((ACC_ROWS, 16), jnp.float32),
      ],
      mesh=mesh,
      compiler_params=_SC_PARAMS,
      scratch_types=[
          pltpu.VMEM_SHARED((ACC_ROWS, 16), jnp.float32),
          pltpu.VMEM((n_chunks // 2, CHUNK), jnp.int32),
          pltpu.VMEM((CHUNK, 16), jnp.float32),
      ],
  )
  def deg_kernel(dst3, ones_hbm, zeros16_hbm, deg0, deg1,
                 deg_sh, didx, ones_v):
    c = lax.axis_index("c")
    s = lax.axis_index("s")
    hc = n_chunks // 2

    pltpu.sync_copy(dst3.at[s, pl.ds(c * hc, hc)], didx)
    pltpu.sync_copy(ones_hbm, ones_v)
    pltpu.sync_copy(zeros16_hbm, deg_sh.at[pl.ds(s * RPT, RPT)])
    plsc.subcore_barrier()

    def body(j, carry):
      pltpu.sync_copy(ones_v, deg_sh.at[didx.at[j]], add=True)
      return carry

    lax.fori_loop(0, hc, body, 0)
    plsc.subcore_barrier()

    @pl.when(c == 0)
    def _():
      pltpu.sync_copy(deg_sh.at[pl.ds(s * RPT, RPT)],
                      deg0.at[pl.ds(s * RPT, RPT)])

    @pl.when(c == 1)
    def _():
      pltpu.sync_copy(deg_sh.at[pl.ds(s * RPT, RPT)],
                      deg1.at[pl.ds(s * RPT, RPT)])

  return deg_kernel


def _row_spec(cols):
  return pl.BlockSpec((ROW_BLK, cols), lambda i: (i, 0))


def _full_spec(shape):
  return pl.BlockSpec(shape, lambda i: (0,) * len(shape))


def _tc_first(x, w_self, w_neigh, b):
  def body(x_ref, ws_ref, wn_ref, b_ref, s_ref, ylo_ref, yhi_ref):
    x = x_ref[...]
    s_ref[...] = (
        jnp.dot(x, ws_ref[...], preferred_element_type=jnp.float32)
        + b_ref[...])
    y = jnp.dot(x, wn_ref[...], preferred_element_type=jnp.float32)
    ylo_ref[...] = y[:, :DH]
    yhi_ref[...] = y[:, DH:]

  n = x.shape[0]
  return pl.pallas_call(
      body,
      grid=(n // ROW_BLK,),
      in_specs=[_row_spec(D), _full_spec((D, D)), _full_spec((D, D)),
                _full_spec((1, D))],
      out_specs=[_row_spec(D), _row_spec(DH), _row_spec(DH)],
      out_shape=[
          jax.ShapeDtypeStruct((n, D), jnp.float32),
          jax.ShapeDtypeStruct((n, DH), jnp.float32),
          jax.ShapeDtypeStruct((n, DH), jnp.float32),
      ],
  )(x, w_self, w_neigh, b)


def _tc_mid(s_prev, a_lo, a_hi, dg0, dg1, w_self, w_neigh, b):
  def body(s_ref, alo_ref, ahi_ref, dg0_ref, dg1_ref, ws_ref, wn_ref, b_ref,
           so_ref, ylo_ref, yhi_ref):
    deg = dg0_ref[...][:, 0:1] + dg1_ref[...][:, 0:1]
    inv = 1.0 / jnp.maximum(deg, 1.0)
    a = jnp.concatenate([alo_ref[...], ahi_ref[...]], axis=1) * inv
    h = jnp.maximum(s_ref[...] + a, 0.0)
    so_ref[...] = (
        jnp.dot(h, ws_ref[...], preferred_element_type=jnp.float32)
        + b_ref[...])
    y = jnp.dot(h, wn_ref[...], preferred_element_type=jnp.float32)
    ylo_ref[...] = y[:, :DH]
    yhi_ref[...] = y[:, DH:]

  n = s_prev.shape[0]
  return pl.pallas_call(
      body,
      grid=(n // ROW_BLK,),
      in_specs=[_row_spec(D), _row_spec(DH), _row_spec(DH), _row_spec(16),
                _row_spec(16), _full_spec((D, D)), _full_spec((D, D)),
                _full_spec((1, D))],
      out_specs=[_row_spec(D), _row_spec(DH), _row_spec(DH)],
      out_shape=[
          jax.ShapeDtypeStruct((n, D), jnp.float32),
          jax.ShapeDtypeStruct((n, DH), jnp.float32),
          jax.ShapeDtypeStruct((n, DH), jnp.float32),
      ],
  )(s_prev, a_lo, a_hi, dg0, dg1, w_self, w_neigh, b)


def _tc_final(s_prev, a_lo, a_hi, dg0, dg1):
  def body(s_ref, alo_ref, ahi_ref, dg0_ref, dg1_ref, out_ref):
    deg = dg0_ref[...][:, 0:1] + dg1_ref[...][:, 0:1]
    inv = 1.0 / jnp.maximum(deg, 1.0)
    a = jnp.concatenate([alo_ref[...], ahi_ref[...]], axis=1) * inv
    out_ref[...] = s_ref[...] + a

  n = s_prev.shape[0]
  return pl.pallas_call(
      body,
      grid=(n // ROW_BLK,),
      in_specs=[_row_spec(D), _row_spec(DH), _row_spec(DH), _row_spec(16),
                _row_spec(16)],
      out_specs=_row_spec(D),
      out_shape=jax.ShapeDtypeStruct((n, D), jnp.float32),
  )(s_prev, a_lo, a_hi, dg0, dg1)


@jax.jit
def kernel(in_feat, edge_index, W_self0, W_neigh0, b0,
           W_self1, W_neigh1, b1, W_self2, W_neigh2, b2):
  n, _ = in_feat.shape
  e = edge_index.shape[1]

  per_tile = -(-e // NT)
  n_chunks = -(-per_tile // CHUNK)
  n_chunks = -(-n_chunks // 4) * 4
  e_pad = NT * n_chunks * CHUNK
  src = jnp.concatenate(
      [edge_index[0], jnp.zeros((e_pad - e,), jnp.int32)])
  dst = jnp.concatenate(
      [edge_index[1], jnp.full((e_pad - e,), SENT, jnp.int32)])
  src3 = src.reshape(NT, n_chunks, CHUNK)
  dst3 = dst.reshape(NT, n_chunks, CHUNK)

  zeros_hbm = jnp.zeros((RPT, DH), jnp.float32)
  zeros16 = jnp.zeros((RPT, 16), jnp.float32)
  ones_hbm = jnp.ones((CHUNK, 16), jnp.float32)

  agg = _make_agg(n_chunks)
  deg_kernel = _make_deg(n_chunks)

  dg0, dg1 = deg_kernel(dst3, ones_hbm, zeros16)

  s0, ylo, yhi = _tc_first(in_feat, W_self0, W_neigh0, b0.reshape(1, D))
  alo, ahi = agg(ylo, yhi, src3, dst3, zeros_hbm)
  s1, ylo, yhi = _tc_mid(s0, alo, ahi, dg0, dg1,
                         W_self1, W_neigh1, b1.reshape(1, D))
  alo, ahi = agg(ylo, yhi, src3, dst3, zeros_hbm)
  s2, ylo, yhi = _tc_mid(s1, alo, ahi, dg0, dg1,
                         W_self2, W_neigh2, b2.reshape(1, D))
  alo, ahi = agg(ylo, yhi, src3, dst3, zeros_hbm)
  return _tc_final(s2, alo, ahi, dg0, dg1)
